# Initial kernel scaffold; baseline (speedup 1.0000x reference)
import jax
import jax.numpy as jnp
from jax.experimental import pallas as pl


def kernel(q, k, topk):
    raise NotImplementedError("write your pallas kernel here")


# speedup vs baseline: 60.1708x; 60.1708x over previous
import jax
import jax.numpy as jnp
from jax.experimental import pallas as pl
from jax.experimental.pallas import tpu as pltpu

_POOL_LB = 128
_BM = 256


def _pool_body(q_ref, k_ref, pq_ref, pk_ref):
    def pool_one(x_ref):
        x = x_ref[0]
        x0 = x[:, 0, :]
        x1 = x[:, 1, :]
        parts = []
        for p in range(4):
            i0 = slice(64 * p, 64 * p + 32)
            i1 = slice(64 * p + 32, 64 * p + 64)
            parts.append(((x0[i0] + x1[i0]) + x0[i1]) + x1[i1])
        return (jnp.concatenate(parts, axis=0) * 0.25).astype(jnp.bfloat16)

    pq_ref[0] = pool_one(q_ref)
    pk_ref[0] = pool_one(k_ref)


def _corr_body(tk_ref, a_ref, b_ref, o_ref):
    a = a_ref[0]
    bm = b_ref[0]
    dn = (((1,), (1,)), ((), ()))
    h = a.shape[1] // 2
    corr = (jax.lax.dot_general(a[:, :h], bm[:, :h], dn,
                                preferred_element_type=jnp.float32)
            + jax.lax.dot_general(a[:, h:], bm[:, h:], dn,
                                  preferred_element_type=jnp.float32))

    kk = tk_ref[0]
    neg = jnp.float32(-jnp.inf)

    def body(_, carry):
        work, s, m = carry
        m = jnp.max(work, axis=1, keepdims=True)
        s = s + m
        work = jnp.where(work >= m, neg, work)
        return work, s, m

    zeros = jnp.zeros((corr.shape[0], 1), jnp.float32)
    _, s, thr = jax.lax.fori_loop(0, kk, body, (corr, zeros, zeros))
    o_ref[0] = jnp.where(corr >= thr, corr, jnp.float32(0.0)) / s


def _corr_topk(pa, pb, tk):
    b, L, D = pa.shape
    return pl.pallas_call(
        _corr_body,
        grid_spec=pltpu.PrefetchScalarGridSpec(
            num_scalar_prefetch=1,
            grid=(b, L // _BM),
            in_specs=[
                pl.BlockSpec((1, _BM, D), lambda bi, gi, tk_s: (bi, gi, 0)),
                pl.BlockSpec((1, L, D), lambda bi, gi, tk_s: (bi, 0, 0)),
            ],
            out_specs=pl.BlockSpec((1, _BM, L), lambda bi, gi, tk_s: (bi, gi, 0)),
        ),
        out_shape=jax.ShapeDtypeStruct((b, L, L), jnp.float32),
    )(tk, pa, pb)


def kernel(q, k, topk):
    b, N, D = q.shape
    L = N // 4
    qr = q.reshape(b, N // 2, 2, D)
    kr = k.reshape(b, N // 2, 2, D)
    pq, pk = pl.pallas_call(
        _pool_body,
        grid=(b, L // _POOL_LB),
        in_specs=[
            pl.BlockSpec((1, 2 * _POOL_LB, 2, D), lambda bi, gi: (bi, gi, 0, 0)),
            pl.BlockSpec((1, 2 * _POOL_LB, 2, D), lambda bi, gi: (bi, gi, 0, 0)),
        ],
        out_specs=[
            pl.BlockSpec((1, _POOL_LB, D), lambda bi, gi: (bi, gi, 0)),
            pl.BlockSpec((1, _POOL_LB, D), lambda bi, gi: (bi, gi, 0)),
        ],
        out_shape=[
            jax.ShapeDtypeStruct((b, L, D), jnp.bfloat16),
            jax.ShapeDtypeStruct((b, L, D), jnp.bfloat16),
        ],
    )(qr, kr)

    tk = jnp.asarray(topk, jnp.int32).reshape(1)
    R = _corr_topk(pq, pk, tk)
    R_ = _corr_topk(pk, pq, tk)
    return (R, R_)
